# baseline (speedup 1.0000x reference)
import functools

import jax
import jax.numpy as jnp
from jax import lax
from jax.experimental import pallas as pl
from jax.experimental.pallas import tpu as pltpu
from jax.experimental.pallas import tpu_sc as plsc

_INFO = plsc.get_sparse_core_info()
_NW = _INFO.num_cores * _INFO.num_subcores


def kernel(img_batch, canvases):
    del canvases
    b, c, h, w = img_batch.shape
    rows = b * c * h
    flat = img_batch.reshape(rows, w)
    rpw = rows // _NW
    mesh = plsc.VectorSubcoreMesh(core_axis_name="c", subcore_axis_name="s")

    @functools.partial(
        pl.kernel,
        out_type=jax.ShapeDtypeStruct((rows, w), jnp.float32),
        mesh=mesh,
        scratch_types=[pltpu.SemaphoreType.DMA],
    )
    def _sc_slot_copy(src_hbm, out_hbm, sem):
        wid = lax.axis_index("s") * _INFO.num_cores + lax.axis_index("c")
        base = wid * rpw
        pltpu.async_copy(
            src_hbm.at[pl.ds(base, rpw)],
            out_hbm.at[pl.ds(base, rpw)],
            sem,
        ).wait()

    return _sc_slot_copy(flat).reshape(b, c, h, w)


# speedup vs baseline: 47.2733x; 47.2733x over previous
import jax
import jax.numpy as jnp
from jax.experimental import pallas as pl
from jax.experimental.pallas import tpu as pltpu

_BLOCK_ROWS = 2048


def _ring_slot_copy(src_ref, dst_ref):
    dst_ref[...] = src_ref[...]


def kernel(img_batch, canvases):
    del canvases
    b, c, h, w = img_batch.shape
    flat = img_batch.reshape(b * c * h, w)
    rows = flat.shape[0]
    grid = rows // _BLOCK_ROWS
    out = pl.pallas_call(
        _ring_slot_copy,
        grid=(grid,),
        in_specs=[pl.BlockSpec((_BLOCK_ROWS, w), lambda i: (i, 0))],
        out_specs=pl.BlockSpec((_BLOCK_ROWS, w), lambda i: (i, 0)),
        out_shape=jax.ShapeDtypeStruct(flat.shape, flat.dtype),
        compiler_params=pltpu.CompilerParams(
            dimension_semantics=("parallel",),
            vmem_limit_bytes=100 * 1024 * 1024,
        ),
    )(flat)
    return out.reshape(b, c, h, w)


# speedup vs baseline: 48.7423x; 1.0311x over previous
import jax
import jax.numpy as jnp
from jax.experimental import pallas as pl
from jax.experimental.pallas import tpu as pltpu

_NCHUNKS = 16


def _ring_slot_copy(src_hbm, dst_hbm, buf, in_sems, out_sems):
    rows = src_hbm.shape[0]
    chunk = rows // _NCHUNKS
    for i in range(_NCHUNKS):
        sl = pl.ds(i * chunk, chunk)
        pltpu.make_async_copy(src_hbm.at[sl], buf.at[sl], in_sems.at[i]).start()
    for i in range(_NCHUNKS):
        sl = pl.ds(i * chunk, chunk)
        pltpu.make_async_copy(src_hbm.at[sl], buf.at[sl], in_sems.at[i]).wait()
        pltpu.make_async_copy(buf.at[sl], dst_hbm.at[sl], out_sems.at[i]).start()
    for i in range(_NCHUNKS):
        sl = pl.ds(i * chunk, chunk)
        pltpu.make_async_copy(buf.at[sl], dst_hbm.at[sl], out_sems.at[i]).wait()


def kernel(img_batch, canvases):
    del canvases
    b, c, h, w = img_batch.shape
    flat = img_batch.reshape(b * c * h, w)
    out = pl.pallas_call(
        _ring_slot_copy,
        in_specs=[pl.BlockSpec(memory_space=pltpu.MemorySpace.HBM)],
        out_specs=pl.BlockSpec(memory_space=pltpu.MemorySpace.HBM),
        out_shape=jax.ShapeDtypeStruct(flat.shape, flat.dtype),
        scratch_shapes=[
            pltpu.VMEM(flat.shape, flat.dtype),
            pltpu.SemaphoreType.DMA((_NCHUNKS,)),
            pltpu.SemaphoreType.DMA((_NCHUNKS,)),
        ],
    )(flat)
    return out.reshape(b, c, h, w)
